# baseline (speedup 1.0000x reference)
import functools

import jax
import jax.numpy as jnp
import numpy as np
from jax import lax
from jax.experimental import pallas as pl
from jax.experimental.pallas import tpu as pltpu
from jax.experimental.pallas import tpu_sc as plsc

N_SYM = 256
D = 2048
B = 16384
EQ_IDX = 0
EPS = 1e-8

NC = 2
NS = 16
L = 16
DSL = D // NS
BH = B // NC
GROUPS = BH // L
FIN = BH // NS

_MAGIC = np.int32(0x5F3759DF)


def _rsqrt(x):
    i = plsc.bitcast(x, jnp.int32)
    i = _MAGIC - (i >> 1)
    y = plsc.bitcast(i, jnp.float32)
    h = 0.5 * x
    for _ in range(3):
        y = y * (1.5 - (h * y) * y)
    return y


def _vsa_body(emb_hbm, a_hbm, b_hbm, op_hbm, r_hbm, out_hbm, part_hbm,
              tab, tab2, ia, ib, iop, ir, p3, tmpb, outb):
    c = lax.axis_index("c")
    s = lax.axis_index("s")

    pltpu.sync_copy(emb_hbm.at[:, pl.ds(s * DSL, DSL)], tab)

    eqv = [tab[EQ_IDX, pl.ds(k * L, L)] for k in range(DSL // L)]

    def scale_row(row, _):
        for k in range(DSL // L):
            tab2[row, pl.ds(k * L, L)] = tab[row, pl.ds(k * L, L)] * eqv[k]
        return _

    lax.fori_loop(0, N_SYM, scale_row, None)

    zero = jnp.zeros((L,), jnp.float32)

    def chunk_loop(ch, _):
        coff = ch * FIN
        pltpu.sync_copy(a_hbm.at[pl.ds(c * BH + coff, FIN)], ia)
        pltpu.sync_copy(b_hbm.at[pl.ds(c * BH + coff, FIN)], ib)
        pltpu.sync_copy(op_hbm.at[pl.ds(c * BH + coff, FIN)], iop)
        pltpu.sync_copy(r_hbm.at[pl.ds(c * BH + coff, FIN)], ir)

        def group(g, _g):
            base = g * L
            av = ia[pl.ds(base, L)]
            bv = ib[pl.ds(base, L)]
            ov = iop[pl.ds(base, L)]
            rv = ir[pl.ds(base, L)]
            num = zero
            bn = zero
            rn = zero
            for col in range(DSL):
                cv = jnp.full((L,), col, jnp.int32)
                a = plsc.load_gather(tab, [av, cv])
                b = plsc.load_gather(tab, [bv, cv])
                o = plsc.load_gather(tab2, [ov, cv])
                r = plsc.load_gather(tab, [rv, cv])
                m = (a * b) * o
                num = num + m * r
                bn = bn + m * m
                rn = rn + r * r
            p3[0, pl.ds(base, L)] = num
            p3[1, pl.ds(base, L)] = bn
            p3[2, pl.ds(base, L)] = rn
            return _g

        lax.fori_loop(0, FIN // L, group, None)
        pltpu.sync_copy(p3, part_hbm.at[c, ch, s])
        return _

    lax.fori_loop(0, NS, chunk_loop, None)
    plsc.subcore_barrier()

    pltpu.sync_copy(part_hbm.at[c, s], tmpb)
    for k in range(FIN // L):
        sl = pl.ds(k * L, L)
        for j in range(3):
            acc = tmpb[0, j, sl]
            for t in range(1, NS):
                acc = acc + tmpb[t, j, sl]
            p3[j, sl] = acc

    for k in range(FIN // L):
        sl = pl.ds(k * L, L)
        num = p3[0, sl]
        bn = p3[1, sl]
        rn = p3[2, sl]
        d1 = jnp.maximum(bn * _rsqrt(bn), EPS)
        d2 = jnp.maximum(rn * _rsqrt(rn), EPS)
        outb[sl] = 1.0 - num / (d1 * d2)

    pltpu.sync_copy(outb, out_hbm.at[pl.ds(c * BH + s * FIN, FIN)])


@jax.jit
def _vsa(embeddings, a_idx, b_idx, op_idx, result_idx):
    mesh = plsc.VectorSubcoreMesh(core_axis_name="c", subcore_axis_name="s")
    fn = functools.partial(
        pl.kernel,
        out_type=(
            jax.ShapeDtypeStruct((B,), jnp.float32),
            jax.ShapeDtypeStruct((NC, NS, NS, 3, FIN), jnp.float32),
        ),
        mesh=mesh,
        compiler_params=pltpu.CompilerParams(needs_layout_passes=False),
        scratch_types=[
            pltpu.VMEM((N_SYM, DSL), jnp.float32),
            pltpu.VMEM((N_SYM, DSL), jnp.float32),
            pltpu.VMEM((FIN,), jnp.int32),
            pltpu.VMEM((FIN,), jnp.int32),
            pltpu.VMEM((FIN,), jnp.int32),
            pltpu.VMEM((FIN,), jnp.int32),
            pltpu.VMEM((3, FIN), jnp.float32),
            pltpu.VMEM((NS, 3, FIN), jnp.float32),
            pltpu.VMEM((FIN,), jnp.float32),
        ],
    )(_vsa_body)
    out, _ = fn(embeddings, a_idx, b_idx, op_idx, result_idx)
    return out


def kernel(embeddings, a_idx, b_idx, op_idx, result_idx):
    return _vsa(embeddings, a_idx, b_idx, op_idx, result_idx)


# speedup vs baseline: 6.9061x; 6.9061x over previous
import functools

import jax
import jax.numpy as jnp
import numpy as np
from jax import lax
from jax.experimental import pallas as pl
from jax.experimental.pallas import tpu as pltpu
from jax.experimental.pallas import tpu_sc as plsc

N_SYM = 256
D = 2048
B = 16384
EQ_IDX = 0
EPS = 1e-8

NC = 2
NS = 16
L = 16
DSL = D // NS
BH = B // NC
GROUPS = BH // L
FIN = BH // NS

_MAGIC = np.int32(0x5F3759DF)


def _rsqrt(x):
    i = plsc.bitcast(x, jnp.int32)
    i = _MAGIC - (i >> 1)
    y = plsc.bitcast(i, jnp.float32)
    h = 0.5 * x
    for _ in range(3):
        y = y * (1.5 - (h * y) * y)
    return y


def _vsa_body(emb_hbm, a_hbm, b_hbm, op_hbm, r_hbm, out_hbm, part_hbm,
              tab, tab2, ia, ib, iop, ir, p3, tmpb, outb):
    c = lax.axis_index("c")
    s = lax.axis_index("s")

    pltpu.sync_copy(emb_hbm.at[:, pl.ds(s * DSL, DSL)], tab)

    eqv = [tab[EQ_IDX, pl.ds(k * L, L)] for k in range(DSL // L)]

    def scale_row(row, _):
        for k in range(DSL // L):
            tab2[row, pl.ds(k * L, L)] = tab[row, pl.ds(k * L, L)] * eqv[k]
        return _

    lax.fori_loop(0, N_SYM, scale_row, None)

    zero = jnp.zeros((L,), jnp.float32)

    def chunk_loop(ch, _):
        coff = ch * FIN
        pltpu.sync_copy(a_hbm.at[pl.ds(c * BH + coff, FIN)], ia)
        pltpu.sync_copy(b_hbm.at[pl.ds(c * BH + coff, FIN)], ib)
        pltpu.sync_copy(op_hbm.at[pl.ds(c * BH + coff, FIN)], iop)
        pltpu.sync_copy(r_hbm.at[pl.ds(c * BH + coff, FIN)], ir)

        lane = lax.iota(jnp.int32, L)

        def group16(g, _g):
            base = g * L
            iav = ia[pl.ds(base, L)]
            ibv = ib[pl.ds(base, L)]
            iov = iop[pl.ds(base, L)]
            irv = ir[pl.ds(base, L)]
            nres = zero
            bres = zero
            rres = zero
            for j in range(L):
                av = iav[j]
                bv = ibv[j]
                ov = iov[j]
                rv = irv[j]
                nacc = zero
                bacc = zero
                racc = zero
                for k in range(DSL // L):
                    sl = pl.ds(k * L, L)
                    a = tab[av, sl]
                    b = tab[bv, sl]
                    o = tab2[ov, sl]
                    r = tab[rv, sl]
                    m = (a * b) * o
                    nacc = nacc + m * r
                    bacc = bacc + m * m
                    racc = racc + r * r
                pick = lane == j
                nres = jnp.where(pick, jnp.sum(nacc), nres)
                bres = jnp.where(pick, jnp.sum(bacc), bres)
                rres = jnp.where(pick, jnp.sum(racc), rres)
            p3[0, pl.ds(base, L)] = nres
            p3[1, pl.ds(base, L)] = bres
            p3[2, pl.ds(base, L)] = rres
            return _g

        lax.fori_loop(0, FIN // L, group16, None)
        pltpu.sync_copy(p3, part_hbm.at[c, ch, s])
        return _

    lax.fori_loop(0, NS, chunk_loop, None)
    plsc.subcore_barrier()

    pltpu.sync_copy(part_hbm.at[c, s], tmpb)
    for k in range(FIN // L):
        sl = pl.ds(k * L, L)
        for j in range(3):
            acc = tmpb[0, j, sl]
            for t in range(1, NS):
                acc = acc + tmpb[t, j, sl]
            p3[j, sl] = acc

    for k in range(FIN // L):
        sl = pl.ds(k * L, L)
        num = p3[0, sl]
        bn = p3[1, sl]
        rn = p3[2, sl]
        d1 = jnp.maximum(bn * _rsqrt(bn), EPS)
        d2 = jnp.maximum(rn * _rsqrt(rn), EPS)
        outb[sl] = 1.0 - num / (d1 * d2)

    pltpu.sync_copy(outb, out_hbm.at[pl.ds(c * BH + s * FIN, FIN)])


@jax.jit
def _vsa(embeddings, a_idx, b_idx, op_idx, result_idx):
    mesh = plsc.VectorSubcoreMesh(core_axis_name="c", subcore_axis_name="s")
    fn = functools.partial(
        pl.kernel,
        out_type=(
            jax.ShapeDtypeStruct((B,), jnp.float32),
            jax.ShapeDtypeStruct((NC, NS, NS, 3, FIN), jnp.float32),
        ),
        mesh=mesh,
        compiler_params=pltpu.CompilerParams(needs_layout_passes=False),
        scratch_types=[
            pltpu.VMEM((N_SYM, DSL), jnp.float32),
            pltpu.VMEM((N_SYM, DSL), jnp.float32),
            pltpu.VMEM((FIN,), jnp.int32),
            pltpu.VMEM((FIN,), jnp.int32),
            pltpu.VMEM((FIN,), jnp.int32),
            pltpu.VMEM((FIN,), jnp.int32),
            pltpu.VMEM((3, FIN), jnp.float32),
            pltpu.VMEM((NS, 3, FIN), jnp.float32),
            pltpu.VMEM((FIN,), jnp.float32),
        ],
    )(_vsa_body)
    out, _ = fn(embeddings, a_idx, b_idx, op_idx, result_idx)
    return out


def kernel(embeddings, a_idx, b_idx, op_idx, result_idx):
    return _vsa(embeddings, a_idx, b_idx, op_idx, result_idx)


# speedup vs baseline: 11.3963x; 1.6502x over previous
import functools

import jax
import jax.numpy as jnp
import numpy as np
from jax import lax
from jax.experimental import pallas as pl
from jax.experimental.pallas import tpu as pltpu
from jax.experimental.pallas import tpu_sc as plsc

N_SYM = 256
D = 2048
B = 16384
EQ_IDX = 0
EPS = 1e-8

NC = 2
NS = 16
L = 16
DSL = D // NS
BH = B // NC
GROUPS = BH // L
FIN = BH // NS

_MAGIC = np.int32(0x5F3759DF)


def _rsqrt(x):
    i = plsc.bitcast(x, jnp.int32)
    i = _MAGIC - (i >> 1)
    y = plsc.bitcast(i, jnp.float32)
    h = 0.5 * x
    for _ in range(3):
        y = y * (1.5 - (h * y) * y)
    return y


def _vsa_body(emb_hbm, a_hbm, b_hbm, op_hbm, r_hbm, out_hbm, part_hbm,
              tab, tabb, tab2b, ia, ib, iop, ir, p3, tmpb, outb):
    c = lax.axis_index("c")
    s = lax.axis_index("s")

    pltpu.sync_copy(emb_hbm.at[:, pl.ds(s * DSL, DSL)], tab)

    eqv = [tab[EQ_IDX, pl.ds(k * L, L)] for k in range(DSL // L)]

    def build_row(row, _):
        for k in range(DSL // (2 * L)):
            x0 = tab[row, pl.ds(2 * k * L, L)]
            x1 = tab[row, pl.ds((2 * k + 1) * L, L)]
            sl32 = pl.ds(2 * k * L, 2 * L)
            tabb[row, sl32] = plsc.pack(
                x0, x1, format=plsc.PackFormat.INTERLEAVED)
            tab2b[row, sl32] = plsc.pack(
                x0 * eqv[2 * k], x1 * eqv[2 * k + 1],
                format=plsc.PackFormat.INTERLEAVED)
        return _

    lax.fori_loop(0, N_SYM, build_row, None)

    zero = jnp.zeros((L,), jnp.float32)

    def chunk_loop(ch, _):
        coff = ch * FIN
        pltpu.sync_copy(a_hbm.at[pl.ds(c * BH + coff, FIN)], ia)
        pltpu.sync_copy(b_hbm.at[pl.ds(c * BH + coff, FIN)], ib)
        pltpu.sync_copy(op_hbm.at[pl.ds(c * BH + coff, FIN)], iop)
        pltpu.sync_copy(r_hbm.at[pl.ds(c * BH + coff, FIN)], ir)

        lane = lax.iota(jnp.int32, L)
        bzero = jnp.zeros((2 * L,), jnp.bfloat16)

        def group16(g, _g):
            base = g * L
            iav = ia[pl.ds(base, L)]
            ibv = ib[pl.ds(base, L)]
            iov = iop[pl.ds(base, L)]
            irv = ir[pl.ds(base, L)]
            nres = zero
            bres = zero
            rres = zero
            for j in range(L):
                av = iav[j]
                bv = ibv[j]
                ov = iov[j]
                rv = irv[j]
                nacc = bzero
                bacc = bzero
                racc = bzero
                for k in range(DSL // (2 * L)):
                    sl = pl.ds(2 * k * L, 2 * L)
                    a = tabb[av, sl]
                    b = tabb[bv, sl]
                    o = tab2b[ov, sl]
                    r = tabb[rv, sl]
                    m = (a * b) * o
                    nacc = nacc + m * r
                    bacc = bacc + m * m
                    racc = racc + r * r
                n0, n1 = plsc.unpack(nacc, format=plsc.PackFormat.INTERLEAVED)
                b0, b1 = plsc.unpack(bacc, format=plsc.PackFormat.INTERLEAVED)
                r0, r1 = plsc.unpack(racc, format=plsc.PackFormat.INTERLEAVED)
                pick = lane == j
                nres = jnp.where(pick, jnp.sum(n0 + n1), nres)
                bres = jnp.where(pick, jnp.sum(b0 + b1), bres)
                rres = jnp.where(pick, jnp.sum(r0 + r1), rres)
            p3[0, pl.ds(base, L)] = nres
            p3[1, pl.ds(base, L)] = bres
            p3[2, pl.ds(base, L)] = rres
            return _g

        lax.fori_loop(0, FIN // L, group16, None)
        pltpu.sync_copy(p3, part_hbm.at[c, ch, s])
        return _

    lax.fori_loop(0, NS, chunk_loop, None)
    plsc.subcore_barrier()

    pltpu.sync_copy(part_hbm.at[c, s], tmpb)
    for k in range(FIN // L):
        sl = pl.ds(k * L, L)
        for j in range(3):
            acc = tmpb[0, j, sl]
            for t in range(1, NS):
                acc = acc + tmpb[t, j, sl]
            p3[j, sl] = acc

    for k in range(FIN // L):
        sl = pl.ds(k * L, L)
        num = p3[0, sl]
        bn = p3[1, sl]
        rn = p3[2, sl]
        d1 = jnp.maximum(bn * _rsqrt(bn), EPS)
        d2 = jnp.maximum(rn * _rsqrt(rn), EPS)
        outb[sl] = 1.0 - num / (d1 * d2)

    pltpu.sync_copy(outb, out_hbm.at[pl.ds(c * BH + s * FIN, FIN)])


@jax.jit
def _vsa(embeddings, a_idx, b_idx, op_idx, result_idx):
    mesh = plsc.VectorSubcoreMesh(core_axis_name="c", subcore_axis_name="s")
    fn = functools.partial(
        pl.kernel,
        out_type=(
            jax.ShapeDtypeStruct((B,), jnp.float32),
            jax.ShapeDtypeStruct((NC, NS, NS, 3, FIN), jnp.float32),
        ),
        mesh=mesh,
        compiler_params=pltpu.CompilerParams(needs_layout_passes=False),
        scratch_types=[
            pltpu.VMEM((N_SYM, DSL), jnp.float32),
            pltpu.VMEM((N_SYM, DSL), jnp.bfloat16),
            pltpu.VMEM((N_SYM, DSL), jnp.bfloat16),
            pltpu.VMEM((FIN,), jnp.int32),
            pltpu.VMEM((FIN,), jnp.int32),
            pltpu.VMEM((FIN,), jnp.int32),
            pltpu.VMEM((FIN,), jnp.int32),
            pltpu.VMEM((3, FIN), jnp.float32),
            pltpu.VMEM((NS, 3, FIN), jnp.float32),
            pltpu.VMEM((FIN,), jnp.float32),
        ],
    )(_vsa_body)
    out, _ = fn(embeddings, a_idx, b_idx, op_idx, result_idx)
    return out


def kernel(embeddings, a_idx, b_idx, op_idx, result_idx):
    return _vsa(embeddings, a_idx, b_idx, op_idx, result_idx)


# speedup vs baseline: 11.4634x; 1.0059x over previous
import functools

import jax
import jax.numpy as jnp
import numpy as np
from jax import lax
from jax.experimental import pallas as pl
from jax.experimental.pallas import tpu as pltpu
from jax.experimental.pallas import tpu_sc as plsc

N_SYM = 256
D = 2048
B = 16384
EQ_IDX = 0
EPS = 1e-8

NC = 2
NS = 16
L = 16
DSL = D // NS
BH = B // NC
GROUPS = BH // L
FIN = BH // NS

_MAGIC = np.int32(0x5F3759DF)


def _rsqrt(x):
    i = plsc.bitcast(x, jnp.int32)
    i = _MAGIC - (i >> 1)
    y = plsc.bitcast(i, jnp.float32)
    h = 0.5 * x
    for _ in range(3):
        y = y * (1.5 - (h * y) * y)
    return y


def _vsa_body(emb_hbm, a_hbm, b_hbm, op_hbm, r_hbm, out_hbm, part_hbm,
              tab, tabb, tab2b, ia, ib, iop, ir, p3, tmpb, outb):
    c = lax.axis_index("c")
    s = lax.axis_index("s")

    pltpu.sync_copy(emb_hbm.at[:, pl.ds(s * DSL, DSL)], tab)

    eqv = [tab[EQ_IDX, pl.ds(k * L, L)] for k in range(DSL // L)]

    PW = DSL // 2

    def build_row(row, _):
        for k in range(DSL // (2 * L)):
            x0 = tab[row, pl.ds(2 * k * L, L)]
            x1 = tab[row, pl.ds((2 * k + 1) * L, L)]
            sl16 = pl.ds(row * PW + k * L, L)
            tabb[sl16] = plsc.bitcast(
                plsc.pack(x0, x1, format=plsc.PackFormat.INTERLEAVED),
                jnp.int32)
            tab2b[sl16] = plsc.bitcast(
                plsc.pack(x0 * eqv[2 * k], x1 * eqv[2 * k + 1],
                          format=plsc.PackFormat.INTERLEAVED),
                jnp.int32)
        return _

    lax.fori_loop(0, N_SYM, build_row, None)

    zero = jnp.zeros((L,), jnp.float32)

    def chunk_loop(ch, _):
        coff = ch * FIN
        pltpu.sync_copy(a_hbm.at[pl.ds(c * BH + coff, FIN)], ia)
        pltpu.sync_copy(b_hbm.at[pl.ds(c * BH + coff, FIN)], ib)
        pltpu.sync_copy(op_hbm.at[pl.ds(c * BH + coff, FIN)], iop)
        pltpu.sync_copy(r_hbm.at[pl.ds(c * BH + coff, FIN)], ir)

        lane = lax.iota(jnp.int32, L)
        bzero = jnp.zeros((2 * L,), jnp.bfloat16)

        def group16(g, _g):
            base = g * L
            iav = ia[pl.ds(base, L)]
            ibv = ib[pl.ds(base, L)]
            iov = iop[pl.ds(base, L)]
            irv = ir[pl.ds(base, L)]
            nres = zero
            bres = zero
            rres = zero
            for j in range(L):
                av = iav[j] * PW
                bv = ibv[j] * PW
                ov = iov[j] * PW
                rv = irv[j] * PW
                nacc = bzero
                bacc = bzero
                racc = bzero
                for k in range(DSL // (2 * L)):
                    a = plsc.bitcast(tabb[pl.ds(av + k * L, L)], jnp.bfloat16)
                    b = plsc.bitcast(tabb[pl.ds(bv + k * L, L)], jnp.bfloat16)
                    o = plsc.bitcast(tab2b[pl.ds(ov + k * L, L)], jnp.bfloat16)
                    r = plsc.bitcast(tabb[pl.ds(rv + k * L, L)], jnp.bfloat16)
                    m = (a * b) * o
                    nacc = nacc + m * r
                    bacc = bacc + m * m
                    racc = racc + r * r
                n0, n1 = plsc.unpack(nacc, format=plsc.PackFormat.INTERLEAVED)
                b0, b1 = plsc.unpack(bacc, format=plsc.PackFormat.INTERLEAVED)
                r0, r1 = plsc.unpack(racc, format=plsc.PackFormat.INTERLEAVED)
                pick = lane == j
                nres = jnp.where(pick, jnp.sum(n0 + n1), nres)
                bres = jnp.where(pick, jnp.sum(b0 + b1), bres)
                rres = jnp.where(pick, jnp.sum(r0 + r1), rres)
            p3[0, pl.ds(base, L)] = nres
            p3[1, pl.ds(base, L)] = bres
            p3[2, pl.ds(base, L)] = rres
            return _g

        lax.fori_loop(0, FIN // L, group16, None)
        pltpu.sync_copy(p3, part_hbm.at[c, ch, s])
        return _

    lax.fori_loop(0, NS, chunk_loop, None)
    plsc.subcore_barrier()

    pltpu.sync_copy(part_hbm.at[c, s], tmpb)
    for k in range(FIN // L):
        sl = pl.ds(k * L, L)
        for j in range(3):
            acc = tmpb[0, j, sl]
            for t in range(1, NS):
                acc = acc + tmpb[t, j, sl]
            p3[j, sl] = acc

    for k in range(FIN // L):
        sl = pl.ds(k * L, L)
        num = p3[0, sl]
        bn = p3[1, sl]
        rn = p3[2, sl]
        d1 = jnp.maximum(bn * _rsqrt(bn), EPS)
        d2 = jnp.maximum(rn * _rsqrt(rn), EPS)
        outb[sl] = 1.0 - num / (d1 * d2)

    pltpu.sync_copy(outb, out_hbm.at[pl.ds(c * BH + s * FIN, FIN)])


@jax.jit
def _vsa(embeddings, a_idx, b_idx, op_idx, result_idx):
    mesh = plsc.VectorSubcoreMesh(core_axis_name="c", subcore_axis_name="s")
    fn = functools.partial(
        pl.kernel,
        out_type=(
            jax.ShapeDtypeStruct((B,), jnp.float32),
            jax.ShapeDtypeStruct((NC, NS, NS, 3, FIN), jnp.float32),
        ),
        mesh=mesh,
        compiler_params=pltpu.CompilerParams(needs_layout_passes=False),
        scratch_types=[
            pltpu.VMEM((N_SYM, DSL), jnp.float32),
            pltpu.VMEM((N_SYM * DSL // 2,), jnp.int32),
            pltpu.VMEM((N_SYM * DSL // 2,), jnp.int32),
            pltpu.VMEM((FIN,), jnp.int32),
            pltpu.VMEM((FIN,), jnp.int32),
            pltpu.VMEM((FIN,), jnp.int32),
            pltpu.VMEM((FIN,), jnp.int32),
            pltpu.VMEM((3, FIN), jnp.float32),
            pltpu.VMEM((NS, 3, FIN), jnp.float32),
            pltpu.VMEM((FIN,), jnp.float32),
        ],
    )(_vsa_body)
    out, _ = fn(embeddings, a_idx, b_idx, op_idx, result_idx)
    return out


def kernel(embeddings, a_idx, b_idx, op_idx, result_idx):
    return _vsa(embeddings, a_idx, b_idx, op_idx, result_idx)


# speedup vs baseline: 14.5090x; 1.2657x over previous
import functools

import jax
import jax.numpy as jnp
import numpy as np
from jax import lax
from jax.experimental import pallas as pl
from jax.experimental.pallas import tpu as pltpu
from jax.experimental.pallas import tpu_sc as plsc

N_SYM = 256
D = 2048
B = 16384
EQ_IDX = 0
EPS = 1e-8

NC = 2
NS = 16
L = 16
DSL = D // NS
BH = B // NC
GROUPS = BH // L
FIN = BH // NS

_MAGIC = np.int32(0x5F3759DF)


def _rsqrt(x):
    i = plsc.bitcast(x, jnp.int32)
    i = _MAGIC - (i >> 1)
    y = plsc.bitcast(i, jnp.float32)
    h = 0.5 * x
    for _ in range(3):
        y = y * (1.5 - (h * y) * y)
    return y


def _vsa_body(emb_hbm, a_hbm, b_hbm, op_hbm, r_hbm, out_hbm, part_hbm,
              tab, tabb, tab2b, ia, ib, iop, ir, p3, tmpb, outb,
              sem_i, sem_p):
    c = lax.axis_index("c")
    s = lax.axis_index("s")

    pltpu.sync_copy(emb_hbm.at[:, pl.ds(s * DSL, DSL)], tab)

    eqv = [tab[EQ_IDX, pl.ds(k * L, L)] for k in range(DSL // L)]

    PW = DSL // 2

    def build_row(row, _):
        for k in range(DSL // (2 * L)):
            x0 = tab[row, pl.ds(2 * k * L, L)]
            x1 = tab[row, pl.ds((2 * k + 1) * L, L)]
            sl16 = pl.ds(row * PW + k * L, L)
            tabb[sl16] = plsc.bitcast(
                plsc.pack(x0, x1, format=plsc.PackFormat.INTERLEAVED),
                jnp.int32)
            tab2b[sl16] = plsc.bitcast(
                plsc.pack(x0 * eqv[2 * k], x1 * eqv[2 * k + 1],
                          format=plsc.PackFormat.INTERLEAVED),
                jnp.int32)
        return _

    lax.fori_loop(0, N_SYM, build_row, None)

    zero = jnp.zeros((L,), jnp.float32)
    idx_bufs = (ia, ib, iop, ir)
    idx_srcs = (a_hbm, b_hbm, op_hbm, r_hbm)

    for src, buf in zip(idx_srcs, idx_bufs):
        pltpu.async_copy(src.at[pl.ds(c * BH, FIN)], buf.at[0], sem_i)

    def chunk_loop(ch, _):
        cur = lax.rem(ch, 2)
        for src, buf in zip(idx_srcs, idx_bufs):
            pltpu.make_async_copy(src.at[pl.ds(0, FIN)], buf.at[0],
                                  sem_i).wait()

        @pl.when(ch < NS - 1)
        def _fire():
            coff2 = (ch + 1) * FIN
            for src, buf in zip(idx_srcs, idx_bufs):
                pltpu.async_copy(src.at[pl.ds(c * BH + coff2, FIN)],
                                 buf.at[1 - cur], sem_i)

        @pl.when(ch >= 2)
        def _drainp():
            pltpu.make_async_copy(p3.at[0], part_hbm.at[c, 0, s],
                                  sem_p).wait()

        lane = lax.iota(jnp.int32, L)
        bzero = jnp.zeros((2 * L,), jnp.bfloat16)

        def group16(g, _g):
            base = g * L
            iav = ia[cur, pl.ds(base, L)]
            ibv = ib[cur, pl.ds(base, L)]
            iov = iop[cur, pl.ds(base, L)]
            irv = ir[cur, pl.ds(base, L)]
            nres = zero
            bres = zero
            rres = zero
            for j in range(L):
                av = iav[j] * PW
                bv = ibv[j] * PW
                ov = iov[j] * PW
                rv = irv[j] * PW
                nacc = bzero
                bacc = bzero
                racc = bzero
                for k in range(DSL // (2 * L)):
                    a = plsc.bitcast(tabb[pl.ds(av + k * L, L)], jnp.bfloat16)
                    b = plsc.bitcast(tabb[pl.ds(bv + k * L, L)], jnp.bfloat16)
                    o = plsc.bitcast(tab2b[pl.ds(ov + k * L, L)], jnp.bfloat16)
                    r = plsc.bitcast(tabb[pl.ds(rv + k * L, L)], jnp.bfloat16)
                    m = (a * b) * o
                    nacc = nacc + m * r
                    bacc = bacc + m * m
                    racc = racc + r * r
                n0, n1 = plsc.unpack(nacc, format=plsc.PackFormat.INTERLEAVED)
                b0, b1 = plsc.unpack(bacc, format=plsc.PackFormat.INTERLEAVED)
                r0, r1 = plsc.unpack(racc, format=plsc.PackFormat.INTERLEAVED)
                pick = lane == j
                nres = jnp.where(pick, jnp.sum(n0 + n1), nres)
                bres = jnp.where(pick, jnp.sum(b0 + b1), bres)
                rres = jnp.where(pick, jnp.sum(r0 + r1), rres)
            p3[cur, 0, pl.ds(base, L)] = nres
            p3[cur, 1, pl.ds(base, L)] = bres
            p3[cur, 2, pl.ds(base, L)] = rres
            return _g

        lax.fori_loop(0, FIN // L, group16, None)
        pltpu.async_copy(p3.at[cur], part_hbm.at[c, ch, s], sem_p)
        return _

    lax.fori_loop(0, NS, chunk_loop, None)
    pltpu.make_async_copy(p3.at[0], part_hbm.at[c, 0, s], sem_p).wait()
    pltpu.make_async_copy(p3.at[0], part_hbm.at[c, 0, s], sem_p).wait()
    plsc.subcore_barrier()

    pltpu.sync_copy(part_hbm.at[c, s], tmpb)
    for k in range(FIN // L):
        sl = pl.ds(k * L, L)
        for j in range(3):
            acc = tmpb[0, j, sl]
            for t in range(1, NS):
                acc = acc + tmpb[t, j, sl]
            p3[0, j, sl] = acc

    for k in range(FIN // L):
        sl = pl.ds(k * L, L)
        num = p3[0, 0, sl]
        bn = p3[0, 1, sl]
        rn = p3[0, 2, sl]
        d1 = jnp.maximum(bn * _rsqrt(bn), EPS)
        d2 = jnp.maximum(rn * _rsqrt(rn), EPS)
        outb[sl] = 1.0 - num / (d1 * d2)

    pltpu.sync_copy(outb, out_hbm.at[pl.ds(c * BH + s * FIN, FIN)])


@jax.jit
def _vsa(embeddings, a_idx, b_idx, op_idx, result_idx):
    mesh = plsc.VectorSubcoreMesh(core_axis_name="c", subcore_axis_name="s")
    fn = functools.partial(
        pl.kernel,
        out_type=(
            jax.ShapeDtypeStruct((B,), jnp.float32),
            jax.ShapeDtypeStruct((NC, NS, NS, 3, FIN), jnp.float32),
        ),
        mesh=mesh,
        compiler_params=pltpu.CompilerParams(needs_layout_passes=False),
        scratch_types=[
            pltpu.VMEM((N_SYM, DSL), jnp.float32),
            pltpu.VMEM((N_SYM * DSL // 2,), jnp.int32),
            pltpu.VMEM((N_SYM * DSL // 2,), jnp.int32),
            pltpu.VMEM((2, FIN), jnp.int32),
            pltpu.VMEM((2, FIN), jnp.int32),
            pltpu.VMEM((2, FIN), jnp.int32),
            pltpu.VMEM((2, FIN), jnp.int32),
            pltpu.VMEM((2, 3, FIN), jnp.float32),
            pltpu.VMEM((NS, 3, FIN), jnp.float32),
            pltpu.VMEM((FIN,), jnp.float32),
            pltpu.SemaphoreType.DMA,
            pltpu.SemaphoreType.DMA,
        ],
    )(_vsa_body)
    out, _ = fn(embeddings, a_idx, b_idx, op_idx, result_idx)
    return out


def kernel(embeddings, a_idx, b_idx, op_idx, result_idx):
    return _vsa(embeddings, a_idx, b_idx, op_idx, result_idx)
